# Initial kernel scaffold; baseline (speedup 1.0000x reference)
import jax
import jax.numpy as jnp
from jax.experimental import pallas as pl


def kernel(x, curve, levels, pmap):
    raise NotImplementedError("write your pallas kernel here")


# speedup vs baseline: 218.6692x; 218.6692x over previous
import functools

import jax
import jax.numpy as jnp
from jax import lax
from jax.experimental import pallas as pl
from jax.experimental.pallas import tpu as pltpu
from jax.experimental.pallas import tpu_sc as plsc

_B, _C, _H, _W = 16, 3, 64, 64
_N = _H * _W
_L = 8
_P = 64
_R = _B * _C
_M = _C * _N
_NCODE = _L * _L


_NC, _NS = 2, 16
_NW = _NC * _NS
_BPW = _N // _NW


@functools.cache
def _sc_gather_call():
    mesh = plsc.VectorSubcoreMesh(core_axis_name="c", subcore_axis_name="s")

    @functools.partial(
        pl.kernel,
        mesh=mesh,
        compiler_params=pltpu.CompilerParams(use_tc_tiling_on_sc=False),
        out_type=jax.ShapeDtypeStruct((_N, _R), jnp.float32),
        scratch_types=[
            pltpu.VMEM((_BPW,), jnp.int32),
            pltpu.VMEM((_BPW, _R), jnp.float32),
            pltpu.SemaphoreType.DMA,
        ],
    )
    def _sc_gather(table_hbm, idx_hbm, out_hbm, idx_v, rows_v, sem):
        wid = lax.axis_index("s") * _NC + lax.axis_index("c")
        base = wid * _BPW
        pltpu.sync_copy(idx_hbm.at[pl.ds(base, _BPW)], idx_v)
        pltpu.async_copy(table_hbm.at[idx_v], rows_v, sem).wait()
        pltpu.sync_copy(rows_v, out_hbm.at[pl.ds(base, _BPW)])

    return _sc_gather


def _tc_body(g_ref, lev_ref, pmap_ref, pfirst_ref, out_ref):
    g = g_ref[...]

    best = jnp.abs(g - lev_ref[0:1, :])
    sym = jnp.zeros((_B, _M), jnp.int32)
    for j in range(1, _L):
        d = jnp.abs(g - lev_ref[j : j + 1, :])
        m = d < best
        sym = jnp.where(m, j, sym)
        best = jnp.where(m, d, best)

    nxt_s = jnp.concatenate([sym[:, 1:], sym[:, :1]], axis=1)
    codes_s = sym * _L + nxt_s
    col_s = lax.broadcasted_iota(jnp.int32, (_B, _M), 1)
    codes_s = jnp.where(col_s == _M - 1, codes_s[:, 0:1], codes_s)

    pm = pmap_ref[...]
    nxt_p = jnp.concatenate([pm[:, 1:], pm[:, :1]], axis=1)
    codes_p = pm * _L + nxt_p
    col_p = lax.broadcasted_iota(jnp.int32, (_P, _N), 1)
    codes_p = jnp.where(col_p == _N - 1, codes_p[:, 0:1], codes_p)

    kcol_s = lax.broadcasted_iota(jnp.int32, (_B, _NCODE), 1)
    kcol_p = lax.broadcasted_iota(jnp.int32, (_P, _NCODE), 1)
    cnt_s = jnp.zeros((_B, _NCODE), jnp.int32)
    cnt_p = jnp.zeros((_P, _NCODE), jnp.int32)
    for k in range(_NCODE):
        rs = jnp.sum((codes_s == k).astype(jnp.int32), axis=1, keepdims=True)
        cnt_s = cnt_s + jnp.where(kcol_s == k, rs, 0)
        rp = jnp.sum((codes_p == k).astype(jnp.int32), axis=1, keepdims=True)
        cnt_p = cnt_p + jnp.where(kcol_p == k, rp, 0)

    ps = (cnt_s > 0).astype(jnp.float32)
    pp = (cnt_p > 0).astype(jnp.float32)

    cs = jnp.sum(ps, axis=1, keepdims=True)
    ones_row = jnp.ones((1, _NCODE), jnp.float32)
    cp_row = lax.dot_general(
        ones_row, pp, (((1,), (1,)), ((), ())), preferred_element_type=jnp.float32
    )
    inter = lax.dot_general(
        ps, pp, (((1,), (1,)), ((), ())), preferred_element_type=jnp.float32
    )

    s_last = sym[:, _M - 1 : _M]
    kb = s_last * _L + pfirst_ref[0:1, :]
    oh = kb[:, :, None] == lax.broadcasted_iota(jnp.int32, (_B, _P, _NCODE), 2)
    u_s = jnp.sum(jnp.where(oh, ps[:, None, :], 0.0), axis=2)
    u_p = jnp.sum(jnp.where(oh, pp[None, :, :], 0.0), axis=2)
    present_kb = ((u_s + u_p) > 0).astype(jnp.float32)

    csp = cs + cp_row - inter + (1.0 - present_kb)
    mn = jnp.minimum(cs, cp_row)
    mx = jnp.maximum(cs, cp_row)
    out_ref[...] = (csp - mn) / mx


_tc_call = pl.pallas_call(
    _tc_body,
    out_shape=jax.ShapeDtypeStruct((_B, _P), jnp.float32),
)


def kernel(x, curve, levels, pmap):
    table = x.reshape(_B * _C, _N).T
    gathered = _sc_gather_call()(table, curve.astype(jnp.int32))
    g = gathered.reshape(_N, _B, _C).transpose(1, 2, 0).reshape(_B, _M)
    lev_b = jnp.repeat(levels.T, _N, axis=1)
    pmap_flat = pmap.reshape(_P, _N).astype(jnp.int32)
    pfirst = jnp.broadcast_to(pmap_flat[:, 0].reshape(1, _P), (8, _P))
    return _tc_call(g, lev_b, pmap_flat, pfirst)


# speedup vs baseline: 290.2626x; 1.3274x over previous
import functools

import jax
import jax.numpy as jnp
from jax import lax
from jax.experimental import pallas as pl
from jax.experimental.pallas import tpu as pltpu
from jax.experimental.pallas import tpu_sc as plsc

_B, _C, _H, _W = 16, 3, 64, 64
_N = _H * _W
_L = 8
_P = 64
_R = _B * _C
_M = _C * _N
_NCODE = _L * _L


_NC, _NS = 2, 16
_NW = _NC * _NS
_BPW = _N // _NW


@functools.cache
def _sc_gather_call():
    mesh = plsc.VectorSubcoreMesh(core_axis_name="c", subcore_axis_name="s")

    @functools.partial(
        pl.kernel,
        mesh=mesh,
        compiler_params=pltpu.CompilerParams(use_tc_tiling_on_sc=False),
        out_type=jax.ShapeDtypeStruct((_N, _R), jnp.float32),
        scratch_types=[
            pltpu.VMEM((_BPW,), jnp.int32),
            pltpu.VMEM((_BPW, _R), jnp.float32),
            pltpu.SemaphoreType.DMA,
        ],
    )
    def _sc_gather(table_hbm, idx_hbm, out_hbm, idx_v, rows_v, sem):
        wid = lax.axis_index("s") * _NC + lax.axis_index("c")
        base = wid * _BPW
        pltpu.sync_copy(idx_hbm.at[pl.ds(base, _BPW)], idx_v)
        pltpu.async_copy(table_hbm.at[idx_v], rows_v, sem).wait()
        pltpu.sync_copy(rows_v, out_hbm.at[pl.ds(base, _BPW)])

    return _sc_gather


def _or_lanes(v):
    n = v.shape[1]
    if n % 3 == 0:
        t = n // 3
        v = v[:, :t] | v[:, t : 2 * t] | v[:, 2 * t :]
        n = t
    while n > 1:
        h = n // 2
        v = v[:, :h] | v[:, h:]
        n = h
    return v


def _presence(codes):
    r = codes.shape[0]
    sh = codes & 31
    val = jnp.left_shift(1, sh)
    lo = _or_lanes(jnp.where(codes < 32, val, 0))
    hi = _or_lanes(jnp.where(codes >= 32, val, 0))
    klane = lax.broadcasted_iota(jnp.int32, (r, _NCODE), 1)
    src = jnp.where(klane < 32, lo, hi)
    bit = lax.shift_right_logical(src, klane & 31) & 1
    return bit.astype(jnp.float32)


def _tc_body(g_ref, lev_ref, pmap_ref, pfirst_ref, out_ref):
    g = g_ref[...]

    best = jnp.abs(g - lev_ref[0:1, :])
    sym = jnp.zeros((_B, _M), jnp.int32)
    for j in range(1, _L):
        d = jnp.abs(g - lev_ref[j : j + 1, :])
        m = d < best
        sym = jnp.where(m, j, sym)
        best = jnp.where(m, d, best)

    nxt_s = jnp.concatenate([sym[:, 1:], sym[:, :1]], axis=1)
    codes_s = sym * _L + nxt_s
    col_s = lax.broadcasted_iota(jnp.int32, (_B, _M), 1)
    codes_s = jnp.where(col_s == _M - 1, codes_s[:, 0:1], codes_s)

    pm = pmap_ref[...]
    nxt_p = jnp.concatenate([pm[:, 1:], pm[:, :1]], axis=1)
    codes_p = pm * _L + nxt_p
    col_p = lax.broadcasted_iota(jnp.int32, (_P, _N), 1)
    codes_p = jnp.where(col_p == _N - 1, codes_p[:, 0:1], codes_p)

    ps = _presence(codes_s)
    pp = _presence(codes_p)

    cs = jnp.sum(ps, axis=1, keepdims=True)
    ones_row = jnp.ones((1, _NCODE), jnp.float32)
    cp_row = lax.dot_general(
        ones_row, pp, (((1,), (1,)), ((), ())), preferred_element_type=jnp.float32
    )
    inter = lax.dot_general(
        ps, pp, (((1,), (1,)), ((), ())), preferred_element_type=jnp.float32
    )

    s_last = sym[:, _M - 1 : _M]
    kb = s_last * _L + pfirst_ref[0:1, :]
    oh = kb[:, :, None] == lax.broadcasted_iota(jnp.int32, (_B, _P, _NCODE), 2)
    u_s = jnp.sum(jnp.where(oh, ps[:, None, :], 0.0), axis=2)
    u_p = jnp.sum(jnp.where(oh, pp[None, :, :], 0.0), axis=2)
    present_kb = ((u_s + u_p) > 0).astype(jnp.float32)

    csp = cs + cp_row - inter + (1.0 - present_kb)
    mn = jnp.minimum(cs, cp_row)
    mx = jnp.maximum(cs, cp_row)
    out_ref[...] = (csp - mn) / mx


_tc_call = pl.pallas_call(
    _tc_body,
    out_shape=jax.ShapeDtypeStruct((_B, _P), jnp.float32),
)


def kernel(x, curve, levels, pmap):
    table = x.reshape(_B * _C, _N).T
    gathered = _sc_gather_call()(table, curve.astype(jnp.int32))
    g = gathered.reshape(_N, _B, _C).transpose(1, 2, 0).reshape(_B, _M)
    lev_b = jnp.repeat(levels.T, _N, axis=1)
    pmap_flat = pmap.reshape(_P, _N).astype(jnp.int32)
    pfirst = jnp.broadcast_to(pmap_flat[:, 0].reshape(1, _P), (8, _P))
    return _tc_call(g, lev_b, pmap_flat, pfirst)


# speedup vs baseline: 290.6279x; 1.0013x over previous
import functools

import jax
import jax.numpy as jnp
from jax import lax
from jax.experimental import pallas as pl
from jax.experimental.pallas import tpu as pltpu
from jax.experimental.pallas import tpu_sc as plsc

_B, _C, _H, _W = 16, 3, 64, 64
_N = _H * _W
_L = 8
_P = 64
_R = _B * _C
_NCODE = _L * _L

_NC, _NS = 2, 16
_NW = _NC * _NS


@functools.cache
def _sc_gather_call():
    mesh = plsc.VectorSubcoreMesh(core_axis_name="c", subcore_axis_name="s")

    @functools.partial(
        pl.kernel,
        mesh=mesh,
        compiler_params=pltpu.CompilerParams(
            use_tc_tiling_on_sc=False, needs_layout_passes=False
        ),
        out_type=jax.ShapeDtypeStruct((_R, _N), jnp.float32),
        scratch_types=[
            pltpu.VMEM((_N,), jnp.int32),
            pltpu.VMEM((_N,), jnp.float32),
            pltpu.VMEM((_N,), jnp.float32),
        ],
    )
    def _sc_gather(x_hbm, idx_hbm, out_hbm, idx_v, row_v, outrow_v):
        wid = lax.axis_index("s") * _NC + lax.axis_index("c")
        pltpu.sync_copy(idx_hbm, idx_v)
        for base in (0, _NW):
            row = wid + base

            @pl.when(row < _R)
            def _():
                pltpu.sync_copy(x_hbm.at[row], row_v)

                def body(j, carry):
                    idx = idx_v[pl.ds(j * 16, 16)]
                    outrow_v[pl.ds(j * 16, 16)] = plsc.load_gather(row_v, [idx])
                    return carry

                lax.fori_loop(0, _N // 16, body, 0)
                pltpu.sync_copy(outrow_v, out_hbm.at[row])

    return _sc_gather


def _or_lanes(v):
    n = v.shape[1]
    if n % 3 == 0:
        t = n // 3
        v = v[:, :t] | v[:, t : 2 * t] | v[:, 2 * t :]
        n = t
    while n > 1:
        h = n // 2
        v = v[:, :h] | v[:, h:]
        n = h
    return v


def _presence(codes):
    r = codes.shape[0]
    sh = codes & 31
    val = jnp.left_shift(1, sh)
    lo = _or_lanes(jnp.where(codes < 32, val, 0))
    hi = _or_lanes(jnp.where(codes >= 32, val, 0))
    klane = lax.broadcasted_iota(jnp.int32, (r, _NCODE), 1)
    src = jnp.where(klane < 32, lo, hi)
    bit = lax.shift_right_logical(src, klane & 31) & 1
    return bit.astype(jnp.float32)


def _row_codes(sym, width):
    nxt = jnp.concatenate([sym[:, 1:], sym[:, :1]], axis=1)
    codes = sym * _L + nxt
    col = lax.broadcasted_iota(jnp.int32, sym.shape, 1)
    return jnp.where(col == width - 1, codes[:, 0:1], codes)


def _tc_body(g_ref, lev_ref, pmap_ref, out_ref):
    g = g_ref[...]

    best = jnp.abs(g - lev_ref[:, 0:1])
    sym = jnp.zeros((_R, _N), jnp.int32)
    for j in range(1, _L):
        d = jnp.abs(g - lev_ref[:, j : j + 1])
        m = d < best
        sym = jnp.where(m, j, sym)
        best = jnp.where(m, d, best)

    pr = _presence(_row_codes(sym, _N))

    first_col = sym[:, 0:1]
    last_col = sym[:, _N - 1 : _N]
    nxt_first = jnp.concatenate([first_col[1:, :], first_col[:1, :]], axis=0)
    cross = last_col * _L + nxt_first
    riota = lax.broadcasted_iota(jnp.int32, (_R, 1), 0)
    valid = (riota % _C) != _C - 1
    klane48 = lax.broadcasted_iota(jnp.int32, (_R, _NCODE), 1)
    ohc = ((cross == klane48) & valid).astype(jnp.float32)

    r48 = lax.broadcasted_iota(jnp.int32, (_R, _B), 0)
    b48 = lax.broadcasted_iota(jnp.int32, (_R, _B), 1)
    fold = (r48 // _C == b48).astype(jnp.float32)
    cnt_b = lax.dot_general(
        fold, pr + ohc, (((0,), (0,)), ((), ())), preferred_element_type=jnp.float32
    )
    ps = (cnt_b > 0).astype(jnp.float32)

    pm = pmap_ref[...]
    pp = _presence(_row_codes(pm, _N))

    cs = jnp.sum(ps, axis=1, keepdims=True)
    ones_row = jnp.ones((1, _NCODE), jnp.float32)
    cp_row = lax.dot_general(
        ones_row, pp, (((1,), (1,)), ((), ())), preferred_element_type=jnp.float32
    )
    inter = lax.dot_general(
        ps, pp, (((1,), (1,)), ((), ())), preferred_element_type=jnp.float32
    )

    sel_last = (r48 == _C * b48 + (_C - 1)).astype(jnp.float32)
    s_last = lax.dot_general(
        sel_last,
        last_col.astype(jnp.float32),
        (((0,), (0,)), ((), ())),
        preferred_element_type=jnp.float32,
    )
    eye = (
        lax.broadcasted_iota(jnp.int32, (_P, _P), 0)
        == lax.broadcasted_iota(jnp.int32, (_P, _P), 1)
    ).astype(jnp.float32)
    p_first = lax.dot_general(
        pm[:, 0:1].astype(jnp.float32),
        eye,
        (((0,), (0,)), ((), ())),
        preferred_element_type=jnp.float32,
    )
    kb = (s_last * _L + p_first).astype(jnp.int32)
    ki = lax.broadcasted_iota(jnp.int32, (_B, _P, _NCODE), 2)
    oh = kb[:, :, None] == ki
    u_s = jnp.sum(jnp.where(oh, ps[:, None, :], 0.0), axis=2)
    u_p = jnp.sum(jnp.where(oh, pp[None, :, :], 0.0), axis=2)
    present_kb = ((u_s + u_p) > 0).astype(jnp.float32)

    csp = cs + cp_row - inter + (1.0 - present_kb)
    mn = jnp.minimum(cs, cp_row)
    mx = jnp.maximum(cs, cp_row)
    out_ref[...] = (csp - mn) / mx


_tc_call = pl.pallas_call(
    _tc_body,
    out_shape=jax.ShapeDtypeStruct((_B, _P), jnp.float32),
)


def kernel(x, curve, levels, pmap):
    xr = x.reshape(_R, _N)
    g = _sc_gather_call()(xr, curve.astype(jnp.int32))
    lev48 = jnp.tile(levels, (_B, 1))
    pmap_flat = pmap.reshape(_P, _N).astype(jnp.int32)
    return _tc_call(g, lev48, pmap_flat)


# speedup vs baseline: 297.7200x; 1.0244x over previous
import functools

import jax
import jax.numpy as jnp
from jax import lax
from jax.experimental import pallas as pl
from jax.experimental.pallas import tpu as pltpu
from jax.experimental.pallas import tpu_sc as plsc

_B, _C, _H, _W = 16, 3, 64, 64
_N = _H * _W
_L = 8
_P = 64
_R = _B * _C
_NCODE = _L * _L

_NC, _NS = 2, 16
_NW = _NC * _NS


_HALF = _N // 2
_UNROLL = 8


@functools.cache
def _sc_gather_call():
    mesh = plsc.VectorSubcoreMesh(core_axis_name="c", subcore_axis_name="s")

    @functools.partial(
        pl.kernel,
        mesh=mesh,
        compiler_params=pltpu.CompilerParams(
            use_tc_tiling_on_sc=True, needs_layout_passes=False
        ),
        out_type=jax.ShapeDtypeStruct((_R, _N), jnp.float32),
        scratch_types=[
            pltpu.VMEM((_N,), jnp.int32),
            pltpu.VMEM((_N,), jnp.float32),
            pltpu.VMEM((_HALF,), jnp.float32),
        ],
    )
    def _sc_gather(x_hbm, idx_hbm, out_hbm, idx_v, row_v, outhalf_v):
        wid = lax.axis_index("s") * _NC + lax.axis_index("c")
        pltpu.sync_copy(idx_hbm, idx_v)
        for t in range(3):
            h = wid * 3 + t
            row = h // 2
            off = (h % 2) * _HALF
            pltpu.sync_copy(x_hbm.at[row], row_v)

            def body(j, carry):
                for u in range(_UNROLL):
                    s = j * (16 * _UNROLL) + u * 16
                    idx = idx_v[pl.ds(off + s, 16)]
                    outhalf_v[pl.ds(s, 16)] = plsc.load_gather(row_v, [idx])
                return carry

            lax.fori_loop(0, _HALF // (16 * _UNROLL), body, 0)
            pltpu.sync_copy(outhalf_v, out_hbm.at[row, pl.ds(off, _HALF)])

    return _sc_gather


def _or_lanes(v):
    n = v.shape[1]
    if n % 3 == 0:
        t = n // 3
        v = v[:, :t] | v[:, t : 2 * t] | v[:, 2 * t :]
        n = t
    while n > 1:
        h = n // 2
        v = v[:, :h] | v[:, h:]
        n = h
    return v


def _presence(codes):
    r = codes.shape[0]
    sh = codes & 31
    val = jnp.left_shift(1, sh)
    lo = _or_lanes(jnp.where(codes < 32, val, 0))
    hi = _or_lanes(jnp.where(codes >= 32, val, 0))
    klane = lax.broadcasted_iota(jnp.int32, (r, _NCODE), 1)
    src = jnp.where(klane < 32, lo, hi)
    bit = lax.shift_right_logical(src, klane & 31) & 1
    return bit.astype(jnp.float32)


def _row_codes(sym, width):
    nxt = jnp.concatenate([sym[:, 1:], sym[:, :1]], axis=1)
    codes = sym * _L + nxt
    col = lax.broadcasted_iota(jnp.int32, sym.shape, 1)
    return jnp.where(col == width - 1, codes[:, 0:1], codes)


def _tc_body(g_ref, lev_ref, pmap_ref, out_ref):
    g = g_ref[...]

    best = jnp.abs(g - lev_ref[:, 0:1])
    sym = jnp.zeros((_R, _N), jnp.int32)
    for j in range(1, _L):
        d = jnp.abs(g - lev_ref[:, j : j + 1])
        m = d < best
        sym = jnp.where(m, j, sym)
        best = jnp.where(m, d, best)

    pr = _presence(_row_codes(sym, _N))

    first_col = sym[:, 0:1]
    last_col = sym[:, _N - 1 : _N]
    nxt_first = jnp.concatenate([first_col[1:, :], first_col[:1, :]], axis=0)
    cross = last_col * _L + nxt_first
    riota = lax.broadcasted_iota(jnp.int32, (_R, 1), 0)
    valid = (riota % _C) != _C - 1
    klane48 = lax.broadcasted_iota(jnp.int32, (_R, _NCODE), 1)
    ohc = ((cross == klane48) & valid).astype(jnp.float32)

    r48 = lax.broadcasted_iota(jnp.int32, (_R, _B), 0)
    b48 = lax.broadcasted_iota(jnp.int32, (_R, _B), 1)
    fold = (r48 // _C == b48).astype(jnp.float32)
    cnt_b = lax.dot_general(
        fold, pr + ohc, (((0,), (0,)), ((), ())), preferred_element_type=jnp.float32
    )
    ps = (cnt_b > 0).astype(jnp.float32)

    pm = pmap_ref[...]
    pp = _presence(_row_codes(pm, _N))

    cs = jnp.sum(ps, axis=1, keepdims=True)
    ones_row = jnp.ones((1, _NCODE), jnp.float32)
    cp_row = lax.dot_general(
        ones_row, pp, (((1,), (1,)), ((), ())), preferred_element_type=jnp.float32
    )
    inter = lax.dot_general(
        ps, pp, (((1,), (1,)), ((), ())), preferred_element_type=jnp.float32
    )

    sel_last = (r48 == _C * b48 + (_C - 1)).astype(jnp.float32)
    s_last = lax.dot_general(
        sel_last,
        last_col.astype(jnp.float32),
        (((0,), (0,)), ((), ())),
        preferred_element_type=jnp.float32,
    )
    eye = (
        lax.broadcasted_iota(jnp.int32, (_P, _P), 0)
        == lax.broadcasted_iota(jnp.int32, (_P, _P), 1)
    ).astype(jnp.float32)
    p_first = lax.dot_general(
        pm[:, 0:1].astype(jnp.float32),
        eye,
        (((0,), (0,)), ((), ())),
        preferred_element_type=jnp.float32,
    )
    kb = (s_last * _L + p_first).astype(jnp.int32)
    ki = lax.broadcasted_iota(jnp.int32, (_B, _P, _NCODE), 2)
    oh = kb[:, :, None] == ki
    u_s = jnp.sum(jnp.where(oh, ps[:, None, :], 0.0), axis=2)
    u_p = jnp.sum(jnp.where(oh, pp[None, :, :], 0.0), axis=2)
    present_kb = ((u_s + u_p) > 0).astype(jnp.float32)

    csp = cs + cp_row - inter + (1.0 - present_kb)
    mn = jnp.minimum(cs, cp_row)
    mx = jnp.maximum(cs, cp_row)
    out_ref[...] = (csp - mn) / mx


_tc_call = pl.pallas_call(
    _tc_body,
    out_shape=jax.ShapeDtypeStruct((_B, _P), jnp.float32),
)


def kernel(x, curve, levels, pmap):
    xr = x.reshape(_R, _N)
    g = _sc_gather_call()(xr, curve.astype(jnp.int32))
    lev48 = jnp.tile(levels, (_B, 1))
    pmap_flat = pmap.reshape(_P, _N).astype(jnp.int32)
    return _tc_call(g, lev48, pmap_flat)
